# baseline (speedup 1.0000x reference)
import functools

import jax
import jax.numpy as jnp
from jax import lax
from jax.experimental import pallas as pl
from jax.experimental.pallas import tpu as pltpu
from jax.experimental.pallas import tpu_sc as plsc

N = 10000
E = 160000
C = 256
R = 8
H = 4 * C
CH2 = C // 2

NUM_CORES = 2
NUM_SUBCORES = 16
LANES = 16

ECH = 128
CHUNKS = 80
EPT = CHUNKS * ECH
E_PAD = EPT * NUM_SUBCORES

SEG = N * R
SEG_PAD = 81920
NODE_PAD = 16
GE = 80
NROWS = 10160
ZCHUNKS = NROWS // GE
DCHUNKS = N // GE

K1CH = EPT // 8
K1CHUNKS = 8
SCH = 640
NSC = EPT // SCH
GPS = SCH // GE

BN = 5000
NB = N // BN
BNQ = 2000
NBQ = N // BNQ


def _pre_body(x_ref, g_ref, b_ref, wrel_ref, hwa_ref, hwb_ref, h_out_ref,
              h_ref):
    r = pl.program_id(1)

    @pl.when(r == 0)
    def _():
        xb = x_ref[0]
        mu = jnp.mean(xb, axis=-1, keepdims=True)
        var = jnp.mean(jnp.square(xb - mu), axis=-1, keepdims=True)
        h = (xb - mu) / jnp.sqrt(var + 1e-5) * g_ref[...] + b_ref[...]
        h_ref[...] = h.astype(jnp.bfloat16)
        h_out_ref[...] = h

    hw = jnp.dot(h_ref[...], wrel_ref[0].astype(jnp.bfloat16),
                 preferred_element_type=jnp.float32)
    hwa_ref[...] = hw[:, :CH2]
    hwb_ref[...] = hw[:, CH2:]


def _pre(x, ln1_g, ln1_b, W_rel):
    return pl.pallas_call(
        _pre_body,
        grid=(NB, R),
        in_specs=[
            pl.BlockSpec((1, BN, C), lambda i, r: (0, i, 0)),
            pl.BlockSpec((C,), lambda i, r: (0,)),
            pl.BlockSpec((C,), lambda i, r: (0,)),
            pl.BlockSpec((1, C, C), lambda i, r: (r, 0, 0)),
        ],
        out_specs=[
            pl.BlockSpec((BN, CH2), lambda i, r: (r * NB + i, 0)),
            pl.BlockSpec((BN, CH2), lambda i, r: (r * NB + i, 0)),
            pl.BlockSpec((BN, C), lambda i, r: (i, 0)),
        ],
        out_shape=[
            jax.ShapeDtypeStruct((R * N, CH2), jnp.float32),
            jax.ShapeDtypeStruct((R * N, CH2), jnp.float32),
            jax.ShapeDtypeStruct((N, C), jnp.float32),
        ],
        scratch_shapes=[pltpu.VMEM((BN, C), jnp.bfloat16)],
    )(x, ln1_g, ln1_b, W_rel)


def _selfgate_body(h_ref, wself_ref, bconv_ref, wgate_ref, bgate_ref,
                   self_ref, gate_ref):
    hb = h_ref[...].astype(jnp.bfloat16)
    self_ref[...] = (
        jnp.dot(hb, wself_ref[...].astype(jnp.bfloat16),
                preferred_element_type=jnp.float32) + bconv_ref[...])
    gate_ref[...] = (
        jnp.dot(hb, wgate_ref[...].astype(jnp.bfloat16),
                preferred_element_type=jnp.float32) + bgate_ref[...])


def _selfgate(h, W_self, b_conv, W_gate, b_gate):
    return pl.pallas_call(
        _selfgate_body,
        grid=(NB,),
        in_specs=[
            pl.BlockSpec((BN, C), lambda i: (i, 0)),
            pl.BlockSpec((C, C), lambda i: (0, 0)),
            pl.BlockSpec((C,), lambda i: (0,)),
            pl.BlockSpec((C, C), lambda i: (0, 0)),
            pl.BlockSpec((C,), lambda i: (0,)),
        ],
        out_specs=[
            pl.BlockSpec((BN, C), lambda i: (i, 0)),
            pl.BlockSpec((BN, C), lambda i: (i, 0)),
        ],
        out_shape=[
            jax.ShapeDtypeStruct((N, C), jnp.float32),
            jax.ShapeDtypeStruct((N, C), jnp.float32),
        ],
    )(h, W_self, b_conv, W_gate, b_gate)


def _sc_count_body(dst_hbm, et_hbm, scale_hbm, cnt_sh, dstb, etb,
                   onesb, sidx4, scaleb, cbuf, invb, sem, q0, q1, q2, q3):
    qsem = (q0, q1, q2, q3)
    core = lax.axis_index("c")
    sub = lax.axis_index("s")
    spt = SEG_PAD // NUM_SUBCORES

    @pl.when(core == 0)
    def _():
        @pl.loop(0, spt // LANES)
        def _(i):
            cbuf[pl.ds(i * LANES, LANES)] = jnp.zeros((LANES,), jnp.float32)

        pltpu.sync_copy(cbuf, cnt_sh.at[pl.ds(sub * spt, spt)])

        @pl.loop(0, ECH // LANES)
        def _(i):
            onesb[pl.ds(i * LANES, LANES)] = jnp.full((LANES,), 1.0,
                                                      jnp.float32)

        base = sub * EPT
        pltpu.async_copy(dst_hbm.at[pl.ds(base, EPT)], dstb, sem).wait()
        pltpu.async_copy(et_hbm.at[pl.ds(base, EPT)], etb, sem).wait()
        plsc.subcore_barrier()

        def cfill(ci, b):
            @pl.loop(0, ECH // LANES)
            def _(g):
                s = pl.ds(ci * ECH + g * LANES, LANES)
                sidx4[b, pl.ds(g * LANES, LANES)] = dstb[s] * R + etb[s]

        def cwait(b):
            pltpu.make_async_copy(onesb, cnt_sh.at[sidx4.at[b]],
                                  qsem[b]).wait()

        @pl.loop(0, CHUNKS // 4)
        def _(p):
            for b in range(4):
                @pl.when(p >= 1)
                def _():
                    cwait(b)

                cfill(p * 4 + b, b)
                pltpu.async_copy(onesb, cnt_sh.at[sidx4.at[b]], qsem[b],
                                 add=True)

        for b in range(4):
            cwait(b)

        plsc.subcore_barrier()
        pltpu.sync_copy(cnt_sh.at[pl.ds(sub * spt, spt)], cbuf)

        @pl.loop(0, spt // LANES)
        def _(i):
            s = pl.ds(i * LANES, LANES)
            cbuf[s] = 1.0 / jnp.maximum(cbuf[s], 1.0)

        pltpu.sync_copy(cbuf, cnt_sh.at[pl.ds(sub * spt, spt)])
        plsc.subcore_barrier()
        pltpu.sync_copy(cnt_sh, invb)

        @pl.loop(0, K1CHUNKS)
        def _(ci):
            @pl.loop(0, K1CH // LANES)
            def _(g):
                s = pl.ds(ci * K1CH + g * LANES, LANES)
                scaleb[pl.ds(g * LANES, LANES)] = plsc.load_gather(
                    invb, [dstb[s] * R + etb[s]])

            pltpu.async_copy(scaleb,
                             scale_hbm.at[pl.ds(base + ci * K1CH, K1CH)],
                             sem).wait()


def _sc_count(dst_pad, et_pad):
    mesh = plsc.VectorSubcoreMesh(core_axis_name="c", subcore_axis_name="s")
    spt = SEG_PAD // NUM_SUBCORES
    return pl.kernel(
        _sc_count_body,
        out_type=jax.ShapeDtypeStruct((E_PAD,), jnp.float32),
        mesh=mesh,
        scratch_types=[
            pltpu.VMEM_SHARED((SEG_PAD,), jnp.float32),
            pltpu.VMEM((EPT,), jnp.int32),
            pltpu.VMEM((EPT,), jnp.int32),
            pltpu.VMEM((ECH,), jnp.float32),
            pltpu.VMEM((4, ECH), jnp.int32),
            pltpu.VMEM((K1CH,), jnp.float32),
            pltpu.VMEM((spt,), jnp.float32),
            pltpu.VMEM((SEG_PAD,), jnp.float32),
            pltpu.SemaphoreType.DMA,
            pltpu.SemaphoreType.DMA,
            pltpu.SemaphoreType.DMA,
            pltpu.SemaphoreType.DMA,
            pltpu.SemaphoreType.DMA,
        ],
        compiler_params=pltpu.CompilerParams(needs_layout_passes=False),
    )(dst_pad, et_pad)


def _sc_agg_body(hwa_hbm, hwb_hbm, src_hbm, dst_hbm, et_hbm, scale_hbm,
                 out_hbm, upd_sh, srcb, dstb, etb, scaleb, gidx4,
                 didx4, rows4, sem, g0, g1, g2, g3, s0, s1, s2, s3):
    core = lax.axis_index("c")
    sub = lax.axis_index("s")
    gsem = (g0, g1, g2, g3)
    ssem = (s0, s1, s2, s3)

    @pl.loop(0, GE)
    def _(i):
        for j in range(CH2 // LANES):
            rows4[0, i, pl.ds(j * LANES, LANES)] = jnp.zeros((LANES,),
                                                             jnp.float32)

    for k in range(-(-ZCHUNKS // NUM_SUBCORES)):
        ci = sub + k * NUM_SUBCORES

        @pl.when(ci < ZCHUNKS)
        def _():
            pltpu.sync_copy(rows4.at[0], upd_sh.at[pl.ds(ci * GE, GE)])

    plsc.subcore_barrier()

    base = sub * EPT

    def run_edges(table_hbm):
        def fill(go, slot):
            @pl.loop(0, GE // LANES)
            def _(g):
                s = pl.ds(go + g * LANES, LANES)
                d = pl.ds(g * LANES, LANES)
                gidx4[slot, d] = etb[s] * N + srcb[s]
                didx4[slot, d] = dstb[s]

        def start_g(slot):
            pltpu.async_copy(table_hbm.at[gidx4.at[slot]], rows4.at[slot],
                             gsem[slot])

        def wait_g(slot):
            pltpu.make_async_copy(table_hbm.at[gidx4.at[slot]],
                                  rows4.at[slot], gsem[slot]).wait()

        def start_s(slot):
            pltpu.async_copy(rows4.at[slot], upd_sh.at[didx4.at[slot]],
                             ssem[slot], add=True)

        def wait_s(slot):
            pltpu.make_async_copy(rows4.at[slot],
                                  upd_sh.at[didx4.at[slot]],
                                  ssem[slot]).wait()

        def mult(slot, gi):
            go = gi * GE
            rb = rows4.at[slot]

            @pl.loop(0, GE // LANES)
            def _(g):
                sv = scaleb[pl.ds(go + g * LANES, LANES)]
                for k in range(LANES):
                    sc = sv[k]
                    e = g * LANES + k
                    for j in range(CH2 // LANES):
                        s = pl.ds(j * LANES, LANES)
                        rb[e, s] = rb[e, s] * sc

        def step(gi, b):
            wait_g(b)

            @pl.when(gi + 2 < GPS)
            def _():
                sl = (b + 2) % 4

                @pl.when(gi >= 2)
                def _():
                    wait_s(sl)

                fill((gi + 2) * GE, sl)
                start_g(sl)

            mult(b, gi)
            start_s(b)

        def superchunk(off):
            c1 = pltpu.async_copy(src_hbm.at[pl.ds(off, SCH)], srcb, sem)
            c2 = pltpu.async_copy(dst_hbm.at[pl.ds(off, SCH)], dstb, sem)
            c3 = pltpu.async_copy(et_hbm.at[pl.ds(off, SCH)], etb, sem)
            c4 = pltpu.async_copy(scale_hbm.at[pl.ds(off, SCH)], scaleb,
                                  sem)
            c1.wait()
            c2.wait()
            c3.wait()
            c4.wait()

            fill(0, 0)
            start_g(0)
            fill(GE, 1)
            start_g(1)

            @pl.loop(0, GPS // 4)
            def _(p):
                for b in range(4):
                    step(p * 4 + b, b)

            for b in range(4):
                wait_s(b)

        @pl.loop(0, NSC)
        def _(ci):
            superchunk(base + ci * SCH)

    @pl.when(core == 0)
    def _():
        run_edges(hwa_hbm)

    @pl.when(core == 1)
    def _():
        run_edges(hwb_hbm)

    plsc.subcore_barrier()
    for k in range(-(-DCHUNKS // NUM_SUBCORES)):
        ci = sub + k * NUM_SUBCORES

        @pl.when(ci < DCHUNKS)
        def _():
            pltpu.sync_copy(upd_sh.at[pl.ds(ci * GE, GE)], rows4.at[0])
            pltpu.sync_copy(rows4.at[0],
                            out_hbm.at[core].at[pl.ds(ci * GE, GE)])

    tail = N - DCHUNKS * GE

    if tail:
        @pl.when(sub == NUM_SUBCORES - 1)
        def _():
            pltpu.sync_copy(upd_sh.at[pl.ds(DCHUNKS * GE, tail)],
                            rows4.at[0].at[pl.ds(0, tail)])
            pltpu.sync_copy(rows4.at[0].at[pl.ds(0, tail)],
                            out_hbm.at[core].at[pl.ds(DCHUNKS * GE, tail)])


def _sc_aggregate(hwa, hwb, src_pad, dst_pad, et_pad, scale):
    mesh = plsc.VectorSubcoreMesh(core_axis_name="c", subcore_axis_name="s")
    return pl.kernel(
        _sc_agg_body,
        out_type=jax.ShapeDtypeStruct((NUM_CORES, N, CH2), jnp.float32),
        mesh=mesh,
        scratch_types=[
            pltpu.VMEM_SHARED((NROWS, CH2), jnp.float32),
            pltpu.VMEM((SCH,), jnp.int32),
            pltpu.VMEM((SCH,), jnp.int32),
            pltpu.VMEM((SCH,), jnp.int32),
            pltpu.VMEM((SCH,), jnp.float32),
            pltpu.VMEM((4, GE), jnp.int32),
            pltpu.VMEM((4, GE), jnp.int32),
            pltpu.VMEM((4, GE, CH2), jnp.float32),
            pltpu.SemaphoreType.DMA,
            pltpu.SemaphoreType.DMA,
            pltpu.SemaphoreType.DMA,
            pltpu.SemaphoreType.DMA,
            pltpu.SemaphoreType.DMA,
            pltpu.SemaphoreType.DMA,
            pltpu.SemaphoreType.DMA,
            pltpu.SemaphoreType.DMA,
            pltpu.SemaphoreType.DMA,
        ],
        compiler_params=pltpu.CompilerParams(needs_layout_passes=False),
    )(hwa, hwb, src_pad, dst_pad, et_pad, scale)


def _post_body(x_ref, ua_ref, ub_ref, self_ref, gate_ref, wproj_ref,
               bproj_ref, g2_ref, b2_ref, wfc1_ref, bfc1_ref, wfc2_ref,
               bfc2_ref, out_ref):
    upd = jnp.concatenate([ua_ref[0], ub_ref[0]], axis=-1) + self_ref[...]
    gate = jax.nn.sigmoid(gate_ref[...])
    conv = gate * jax.nn.gelu(upd)
    y = x_ref[0] + jnp.dot(conv.astype(jnp.bfloat16),
                           wproj_ref[...].astype(jnp.bfloat16),
                           preferred_element_type=jnp.float32) + bproj_ref[...]
    mu = jnp.mean(y, axis=-1, keepdims=True)
    var = jnp.mean(jnp.square(y - mu), axis=-1, keepdims=True)
    h2 = (y - mu) / jnp.sqrt(var + 1e-5) * g2_ref[...] + b2_ref[...]
    f1 = jnp.dot(h2.astype(jnp.bfloat16), wfc1_ref[...].astype(jnp.bfloat16),
                 preferred_element_type=jnp.float32) + bfc1_ref[...]
    ffn = jnp.dot(
        jax.nn.gelu(f1).astype(jnp.bfloat16),
        wfc2_ref[...].astype(jnp.bfloat16),
        preferred_element_type=jnp.float32) + bfc2_ref[...]
    out_ref[0] = y + ffn


def _post(x, upd, a_self, a_gate, W_proj, b_proj, ln2_g, ln2_b,
          W_fc1, b_fc1, W_fc2, b_fc2):
    return pl.pallas_call(
        _post_body,
        grid=(NBQ,),
        in_specs=[
            pl.BlockSpec((1, BNQ, C), lambda i: (0, i, 0)),
            pl.BlockSpec((1, BNQ, CH2), lambda i: (0, i, 0)),
            pl.BlockSpec((1, BNQ, CH2), lambda i: (1, i, 0)),
            pl.BlockSpec((BNQ, C), lambda i: (i, 0)),
            pl.BlockSpec((BNQ, C), lambda i: (i, 0)),
            pl.BlockSpec((C, C), lambda i: (0, 0)),
            pl.BlockSpec((C,), lambda i: (0,)),
            pl.BlockSpec((C,), lambda i: (0,)),
            pl.BlockSpec((C,), lambda i: (0,)),
            pl.BlockSpec((C, H), lambda i: (0, 0)),
            pl.BlockSpec((H,), lambda i: (0,)),
            pl.BlockSpec((H, C), lambda i: (0, 0)),
            pl.BlockSpec((C,), lambda i: (0,)),
        ],
        out_specs=pl.BlockSpec((1, BNQ, C), lambda i: (0, i, 0)),
        out_shape=jax.ShapeDtypeStruct((1, N, C), jnp.float32),
    )(x, upd, upd, a_self, a_gate, W_proj, b_proj, ln2_g, ln2_b,
      W_fc1, b_fc1, W_fc2, b_fc2)


@jax.jit
def kernel(x, edge_index, edge_type, ln1_g, ln1_b, W_rel, W_self, b_conv,
           W_gate, b_gate, W_proj, b_proj, ln2_g, ln2_b,
           W_fc1, b_fc1, W_fc2, b_fc2):
    src = edge_index[0].astype(jnp.int32)
    dst = edge_index[1].astype(jnp.int32)
    et = edge_type.astype(jnp.int32)

    npad = E_PAD - E
    pad_i = jnp.arange(npad, dtype=jnp.int32)
    src_pad = jnp.concatenate([src, (pad_i * 127) % N])
    dst_pad = jnp.concatenate([dst, N + (pad_i % NODE_PAD)])
    et_pad = jnp.concatenate([et, jnp.zeros((npad,), jnp.int32)])

    hwa, hwb, h = _pre(x, ln1_g, ln1_b, W_rel)
    scale = _sc_count(dst_pad, et_pad)
    upd = _sc_aggregate(hwa, hwb, src_pad, dst_pad, et_pad, scale)
    a_self, a_gate = _selfgate(h, W_self, b_conv, W_gate, b_gate)
    return _post(x, upd, a_self, a_gate, W_proj, b_proj, ln2_g, ln2_b,
                 W_fc1, b_fc1, W_fc2, b_fc2)


# speedup vs baseline: 1.0510x; 1.0510x over previous
import functools

import jax
import jax.numpy as jnp
from jax import lax
from jax.experimental import pallas as pl
from jax.experimental.pallas import tpu as pltpu
from jax.experimental.pallas import tpu_sc as plsc

N = 10000
E = 160000
C = 256
R = 8
H = 4 * C
CH2 = C // 2

NUM_CORES = 2
NUM_SUBCORES = 16
LANES = 16

ECH = 128
CHUNKS = 80
EPT = CHUNKS * ECH
E_PAD = EPT * NUM_SUBCORES

SEG = N * R
SEG_PAD = 81920
NODE_PAD = 16
GE = 64
NROWS = 10112
ZCHUNKS = NROWS // GE
DCHUNKS = N // GE

K1CH = EPT // 8
K1CHUNKS = 8
SCH = 1280
NSC = EPT // SCH
GPS = SCH // GE

BN = 5000
NB = N // BN
BNQ = 2000
NBQ = N // BNQ


def _pre_body(x_ref, g_ref, b_ref, wrel_ref, hwa_ref, hwb_ref, h_out_ref,
              h_ref):
    r = pl.program_id(1)

    @pl.when(r == 0)
    def _():
        xb = x_ref[0]
        mu = jnp.mean(xb, axis=-1, keepdims=True)
        var = jnp.mean(jnp.square(xb - mu), axis=-1, keepdims=True)
        h = (xb - mu) / jnp.sqrt(var + 1e-5) * g_ref[...] + b_ref[...]
        h_ref[...] = h.astype(jnp.bfloat16)
        h_out_ref[...] = h

    hw = jnp.dot(h_ref[...], wrel_ref[0].astype(jnp.bfloat16),
                 preferred_element_type=jnp.float32)
    hwa_ref[...] = hw[:, :CH2]
    hwb_ref[...] = hw[:, CH2:]


def _pre(x, ln1_g, ln1_b, W_rel):
    return pl.pallas_call(
        _pre_body,
        grid=(NB, R),
        in_specs=[
            pl.BlockSpec((1, BN, C), lambda i, r: (0, i, 0)),
            pl.BlockSpec((C,), lambda i, r: (0,)),
            pl.BlockSpec((C,), lambda i, r: (0,)),
            pl.BlockSpec((1, C, C), lambda i, r: (r, 0, 0)),
        ],
        out_specs=[
            pl.BlockSpec((BN, CH2), lambda i, r: (r * NB + i, 0)),
            pl.BlockSpec((BN, CH2), lambda i, r: (r * NB + i, 0)),
            pl.BlockSpec((BN, C), lambda i, r: (i, 0)),
        ],
        out_shape=[
            jax.ShapeDtypeStruct((R * N, CH2), jnp.float32),
            jax.ShapeDtypeStruct((R * N, CH2), jnp.float32),
            jax.ShapeDtypeStruct((N, C), jnp.float32),
        ],
        scratch_shapes=[pltpu.VMEM((BN, C), jnp.bfloat16)],
    )(x, ln1_g, ln1_b, W_rel)


def _selfgate_body(h_ref, wself_ref, bconv_ref, wgate_ref, bgate_ref,
                   self_ref, gate_ref):
    hb = h_ref[...].astype(jnp.bfloat16)
    self_ref[...] = (
        jnp.dot(hb, wself_ref[...].astype(jnp.bfloat16),
                preferred_element_type=jnp.float32) + bconv_ref[...])
    gate_ref[...] = (
        jnp.dot(hb, wgate_ref[...].astype(jnp.bfloat16),
                preferred_element_type=jnp.float32) + bgate_ref[...])


def _selfgate(h, W_self, b_conv, W_gate, b_gate):
    return pl.pallas_call(
        _selfgate_body,
        grid=(NB,),
        in_specs=[
            pl.BlockSpec((BN, C), lambda i: (i, 0)),
            pl.BlockSpec((C, C), lambda i: (0, 0)),
            pl.BlockSpec((C,), lambda i: (0,)),
            pl.BlockSpec((C, C), lambda i: (0, 0)),
            pl.BlockSpec((C,), lambda i: (0,)),
        ],
        out_specs=[
            pl.BlockSpec((BN, C), lambda i: (i, 0)),
            pl.BlockSpec((BN, C), lambda i: (i, 0)),
        ],
        out_shape=[
            jax.ShapeDtypeStruct((N, C), jnp.float32),
            jax.ShapeDtypeStruct((N, C), jnp.float32),
        ],
    )(h, W_self, b_conv, W_gate, b_gate)


def _sc_count_body(dst_hbm, et_hbm, scale_hbm, cnt_sh, dstb, etb,
                   onesb, sidx4, scaleb, cbuf, invb, sem, q0, q1, q2, q3):
    qsem = (q0, q1, q2, q3)
    core = lax.axis_index("c")
    sub = lax.axis_index("s")
    spt = SEG_PAD // NUM_SUBCORES

    @pl.when(core == 0)
    def _():
        @pl.loop(0, spt // LANES)
        def _(i):
            cbuf[pl.ds(i * LANES, LANES)] = jnp.zeros((LANES,), jnp.float32)

        pltpu.sync_copy(cbuf, cnt_sh.at[pl.ds(sub * spt, spt)])

        @pl.loop(0, ECH // LANES)
        def _(i):
            onesb[pl.ds(i * LANES, LANES)] = jnp.full((LANES,), 1.0,
                                                      jnp.float32)

        base = sub * EPT
        pltpu.async_copy(dst_hbm.at[pl.ds(base, EPT)], dstb, sem).wait()
        pltpu.async_copy(et_hbm.at[pl.ds(base, EPT)], etb, sem).wait()
        plsc.subcore_barrier()

        def cfill(ci, b):
            @pl.loop(0, ECH // LANES)
            def _(g):
                s = pl.ds(ci * ECH + g * LANES, LANES)
                sidx4[b, pl.ds(g * LANES, LANES)] = dstb[s] * R + etb[s]

        def cwait(b):
            pltpu.make_async_copy(onesb, cnt_sh.at[sidx4.at[b]],
                                  qsem[b]).wait()

        @pl.loop(0, CHUNKS // 4)
        def _(p):
            for b in range(4):
                @pl.when(p >= 1)
                def _():
                    cwait(b)

                cfill(p * 4 + b, b)
                pltpu.async_copy(onesb, cnt_sh.at[sidx4.at[b]], qsem[b],
                                 add=True)

        for b in range(4):
            cwait(b)

        plsc.subcore_barrier()
        pltpu.sync_copy(cnt_sh.at[pl.ds(sub * spt, spt)], cbuf)

        @pl.loop(0, spt // LANES)
        def _(i):
            s = pl.ds(i * LANES, LANES)
            cbuf[s] = 1.0 / jnp.maximum(cbuf[s], 1.0)

        pltpu.sync_copy(cbuf, cnt_sh.at[pl.ds(sub * spt, spt)])
        plsc.subcore_barrier()
        pltpu.sync_copy(cnt_sh, invb)

        @pl.loop(0, K1CHUNKS)
        def _(ci):
            @pl.loop(0, K1CH // LANES)
            def _(g):
                s = pl.ds(ci * K1CH + g * LANES, LANES)
                scaleb[pl.ds(g * LANES, LANES)] = plsc.load_gather(
                    invb, [dstb[s] * R + etb[s]])

            pltpu.async_copy(scaleb,
                             scale_hbm.at[pl.ds(base + ci * K1CH, K1CH)],
                             sem).wait()


def _sc_count(dst_pad, et_pad):
    mesh = plsc.VectorSubcoreMesh(core_axis_name="c", subcore_axis_name="s")
    spt = SEG_PAD // NUM_SUBCORES
    return pl.kernel(
        _sc_count_body,
        out_type=jax.ShapeDtypeStruct((E_PAD,), jnp.float32),
        mesh=mesh,
        scratch_types=[
            pltpu.VMEM_SHARED((SEG_PAD,), jnp.float32),
            pltpu.VMEM((EPT,), jnp.int32),
            pltpu.VMEM((EPT,), jnp.int32),
            pltpu.VMEM((ECH,), jnp.float32),
            pltpu.VMEM((4, ECH), jnp.int32),
            pltpu.VMEM((K1CH,), jnp.float32),
            pltpu.VMEM((spt,), jnp.float32),
            pltpu.VMEM((SEG_PAD,), jnp.float32),
            pltpu.SemaphoreType.DMA,
            pltpu.SemaphoreType.DMA,
            pltpu.SemaphoreType.DMA,
            pltpu.SemaphoreType.DMA,
            pltpu.SemaphoreType.DMA,
        ],
        compiler_params=pltpu.CompilerParams(needs_layout_passes=False),
    )(dst_pad, et_pad)


def _sc_agg_body(hwa_hbm, hwb_hbm, src_hbm, dst_hbm, et_hbm, scale_hbm,
                 out_hbm, upd_sh, srcb, dstb, etb, scaleb, gidx4,
                 didx4, rows4, sem, g0, g1, g2, g3, s0, s1, s2, s3):
    core = lax.axis_index("c")
    sub = lax.axis_index("s")
    gsem = (g0, g1, g2, g3)
    ssem = (s0, s1, s2, s3)

    @pl.loop(0, GE)
    def _(i):
        for j in range(CH2 // LANES):
            rows4[0, i, pl.ds(j * LANES, LANES)] = jnp.zeros((LANES,),
                                                             jnp.float32)

    for k in range(-(-ZCHUNKS // NUM_SUBCORES)):
        ci = sub + k * NUM_SUBCORES

        @pl.when(ci < ZCHUNKS)
        def _():
            pltpu.sync_copy(rows4.at[0], upd_sh.at[pl.ds(ci * GE, GE)])

    plsc.subcore_barrier()

    base = sub * EPT

    def run_edges(table_hbm):
        def fill(go, slot):
            @pl.loop(0, GE // LANES)
            def _(g):
                s = pl.ds(go + g * LANES, LANES)
                d = pl.ds(g * LANES, LANES)
                gidx4[slot, d] = etb[s] * N + srcb[s]
                didx4[slot, d] = dstb[s]

        def start_g(slot):
            pltpu.async_copy(table_hbm.at[gidx4.at[slot]], rows4.at[slot],
                             gsem[slot])

        def wait_g(slot):
            pltpu.make_async_copy(table_hbm.at[gidx4.at[slot]],
                                  rows4.at[slot], gsem[slot]).wait()

        def start_s(slot):
            pltpu.async_copy(rows4.at[slot], upd_sh.at[didx4.at[slot]],
                             ssem[slot], add=True)

        def wait_s(slot):
            pltpu.make_async_copy(rows4.at[slot],
                                  upd_sh.at[didx4.at[slot]],
                                  ssem[slot]).wait()

        def mult(slot, gi):
            go = gi * GE
            rb = rows4.at[slot]

            @pl.loop(0, GE // LANES)
            def _(g):
                sv = scaleb[pl.ds(go + g * LANES, LANES)]
                for k in range(LANES):
                    sc = sv[k]
                    e = g * LANES + k
                    for j in range(CH2 // LANES):
                        s = pl.ds(j * LANES, LANES)
                        rb[e, s] = rb[e, s] * sc

        def step(gi, b):
            wait_g(b)

            @pl.when(gi + 2 < GPS)
            def _():
                sl = (b + 2) % 4

                @pl.when(gi >= 2)
                def _():
                    wait_s(sl)

                fill((gi + 2) * GE, sl)
                start_g(sl)

            mult(b, gi)
            start_s(b)

        def superchunk(off):
            c1 = pltpu.async_copy(src_hbm.at[pl.ds(off, SCH)], srcb, sem)
            c2 = pltpu.async_copy(dst_hbm.at[pl.ds(off, SCH)], dstb, sem)
            c3 = pltpu.async_copy(et_hbm.at[pl.ds(off, SCH)], etb, sem)
            c4 = pltpu.async_copy(scale_hbm.at[pl.ds(off, SCH)], scaleb,
                                  sem)
            c1.wait()
            c2.wait()
            c3.wait()
            c4.wait()

            fill(0, 0)
            start_g(0)
            fill(GE, 1)
            start_g(1)

            @pl.loop(0, GPS // 4)
            def _(p):
                for b in range(4):
                    step(p * 4 + b, b)

            for b in range(4):
                wait_s(b)

        @pl.loop(0, NSC)
        def _(ci):
            superchunk(base + ci * SCH)

    @pl.when(core == 0)
    def _():
        run_edges(hwa_hbm)

    @pl.when(core == 1)
    def _():
        run_edges(hwb_hbm)

    plsc.subcore_barrier()
    for k in range(-(-DCHUNKS // NUM_SUBCORES)):
        ci = sub + k * NUM_SUBCORES

        @pl.when(ci < DCHUNKS)
        def _():
            pltpu.sync_copy(upd_sh.at[pl.ds(ci * GE, GE)], rows4.at[0])
            pltpu.sync_copy(rows4.at[0],
                            out_hbm.at[core].at[pl.ds(ci * GE, GE)])

    tail = N - DCHUNKS * GE

    @pl.when(sub == NUM_SUBCORES - 1)
    def _():
        pltpu.sync_copy(upd_sh.at[pl.ds(DCHUNKS * GE, tail)],
                        rows4.at[0].at[pl.ds(0, tail)])
        pltpu.sync_copy(rows4.at[0].at[pl.ds(0, tail)],
                        out_hbm.at[core].at[pl.ds(DCHUNKS * GE, tail)])


def _sc_aggregate(hwa, hwb, src_pad, dst_pad, et_pad, scale):
    mesh = plsc.VectorSubcoreMesh(core_axis_name="c", subcore_axis_name="s")
    return pl.kernel(
        _sc_agg_body,
        out_type=jax.ShapeDtypeStruct((NUM_CORES, N, CH2), jnp.float32),
        mesh=mesh,
        scratch_types=[
            pltpu.VMEM_SHARED((NROWS, CH2), jnp.float32),
            pltpu.VMEM((SCH,), jnp.int32),
            pltpu.VMEM((SCH,), jnp.int32),
            pltpu.VMEM((SCH,), jnp.int32),
            pltpu.VMEM((SCH,), jnp.float32),
            pltpu.VMEM((4, GE), jnp.int32),
            pltpu.VMEM((4, GE), jnp.int32),
            pltpu.VMEM((4, GE, CH2), jnp.float32),
            pltpu.SemaphoreType.DMA,
            pltpu.SemaphoreType.DMA,
            pltpu.SemaphoreType.DMA,
            pltpu.SemaphoreType.DMA,
            pltpu.SemaphoreType.DMA,
            pltpu.SemaphoreType.DMA,
            pltpu.SemaphoreType.DMA,
            pltpu.SemaphoreType.DMA,
            pltpu.SemaphoreType.DMA,
        ],
        compiler_params=pltpu.CompilerParams(needs_layout_passes=False),
    )(hwa, hwb, src_pad, dst_pad, et_pad, scale)


def _post_body(x_ref, ua_ref, ub_ref, self_ref, gate_ref, wproj_ref,
               bproj_ref, g2_ref, b2_ref, wfc1_ref, bfc1_ref, wfc2_ref,
               bfc2_ref, out_ref):
    upd = jnp.concatenate([ua_ref[0], ub_ref[0]], axis=-1) + self_ref[...]
    gate = jax.nn.sigmoid(gate_ref[...])
    conv = gate * jax.nn.gelu(upd)
    y = x_ref[0] + jnp.dot(conv.astype(jnp.bfloat16),
                           wproj_ref[...].astype(jnp.bfloat16),
                           preferred_element_type=jnp.float32) + bproj_ref[...]
    mu = jnp.mean(y, axis=-1, keepdims=True)
    var = jnp.mean(jnp.square(y - mu), axis=-1, keepdims=True)
    h2 = (y - mu) / jnp.sqrt(var + 1e-5) * g2_ref[...] + b2_ref[...]
    f1 = jnp.dot(h2.astype(jnp.bfloat16), wfc1_ref[...].astype(jnp.bfloat16),
                 preferred_element_type=jnp.float32) + bfc1_ref[...]
    ffn = jnp.dot(
        jax.nn.gelu(f1).astype(jnp.bfloat16),
        wfc2_ref[...].astype(jnp.bfloat16),
        preferred_element_type=jnp.float32) + bfc2_ref[...]
    out_ref[0] = y + ffn


def _post(x, upd, a_self, a_gate, W_proj, b_proj, ln2_g, ln2_b,
          W_fc1, b_fc1, W_fc2, b_fc2):
    return pl.pallas_call(
        _post_body,
        grid=(NBQ,),
        in_specs=[
            pl.BlockSpec((1, BNQ, C), lambda i: (0, i, 0)),
            pl.BlockSpec((1, BNQ, CH2), lambda i: (0, i, 0)),
            pl.BlockSpec((1, BNQ, CH2), lambda i: (1, i, 0)),
            pl.BlockSpec((BNQ, C), lambda i: (i, 0)),
            pl.BlockSpec((BNQ, C), lambda i: (i, 0)),
            pl.BlockSpec((C, C), lambda i: (0, 0)),
            pl.BlockSpec((C,), lambda i: (0,)),
            pl.BlockSpec((C,), lambda i: (0,)),
            pl.BlockSpec((C,), lambda i: (0,)),
            pl.BlockSpec((C, H), lambda i: (0, 0)),
            pl.BlockSpec((H,), lambda i: (0,)),
            pl.BlockSpec((H, C), lambda i: (0, 0)),
            pl.BlockSpec((C,), lambda i: (0,)),
        ],
        out_specs=pl.BlockSpec((1, BNQ, C), lambda i: (0, i, 0)),
        out_shape=jax.ShapeDtypeStruct((1, N, C), jnp.float32),
    )(x, upd, upd, a_self, a_gate, W_proj, b_proj, ln2_g, ln2_b,
      W_fc1, b_fc1, W_fc2, b_fc2)


@jax.jit
def kernel(x, edge_index, edge_type, ln1_g, ln1_b, W_rel, W_self, b_conv,
           W_gate, b_gate, W_proj, b_proj, ln2_g, ln2_b,
           W_fc1, b_fc1, W_fc2, b_fc2):
    src = edge_index[0].astype(jnp.int32)
    dst = edge_index[1].astype(jnp.int32)
    et = edge_type.astype(jnp.int32)

    npad = E_PAD - E
    pad_i = jnp.arange(npad, dtype=jnp.int32)
    src_pad = jnp.concatenate([src, (pad_i * 127) % N])
    dst_pad = jnp.concatenate([dst, N + (pad_i % NODE_PAD)])
    et_pad = jnp.concatenate([et, jnp.zeros((npad,), jnp.int32)])

    hwa, hwb, h = _pre(x, ln1_g, ln1_b, W_rel)
    scale = _sc_count(dst_pad, et_pad)
    upd = _sc_aggregate(hwa, hwb, src_pad, dst_pad, et_pad, scale)
    a_self, a_gate = _selfgate(h, W_self, b_conv, W_gate, b_gate)
    return _post(x, upd, a_self, a_gate, W_proj, b_proj, ln2_g, ln2_b,
                 W_fc1, b_fc1, W_fc2, b_fc2)


# speedup vs baseline: 1.0823x; 1.0298x over previous
import functools

import jax
import jax.numpy as jnp
from jax import lax
from jax.experimental import pallas as pl
from jax.experimental.pallas import tpu as pltpu
from jax.experimental.pallas import tpu_sc as plsc

N = 10000
E = 160000
C = 256
R = 8
H = 4 * C
CH2 = C // 2

NUM_CORES = 2
NUM_SUBCORES = 16
LANES = 16

ECH = 128
CHUNKS = 80
EPT = CHUNKS * ECH
E_PAD = EPT * NUM_SUBCORES

SEG = N * R
SEG_PAD = 81920
NODE_PAD = 16
GE = 64
NROWS = 10112
ZCHUNKS = NROWS // GE
DCHUNKS = N // GE

K1CH = EPT // 8
K1CHUNKS = 8
SCH = 2560
NSC = EPT // SCH
GPS = SCH // GE

BN = 5000
NB = N // BN
BNQ = 2000
NBQ = N // BNQ


def _pre_body(x_ref, g_ref, b_ref, wrel_ref, hwa_ref, hwb_ref, h_out_ref,
              h_ref):
    r = pl.program_id(1)

    @pl.when(r == 0)
    def _():
        xb = x_ref[0]
        mu = jnp.mean(xb, axis=-1, keepdims=True)
        var = jnp.mean(jnp.square(xb - mu), axis=-1, keepdims=True)
        h = (xb - mu) / jnp.sqrt(var + 1e-5) * g_ref[...] + b_ref[...]
        h_ref[...] = h.astype(jnp.bfloat16)
        h_out_ref[...] = h

    hw = jnp.dot(h_ref[...], wrel_ref[0].astype(jnp.bfloat16),
                 preferred_element_type=jnp.float32)
    hwa_ref[...] = hw[:, :CH2]
    hwb_ref[...] = hw[:, CH2:]


def _pre(x, ln1_g, ln1_b, W_rel):
    return pl.pallas_call(
        _pre_body,
        grid=(NB, R),
        in_specs=[
            pl.BlockSpec((1, BN, C), lambda i, r: (0, i, 0)),
            pl.BlockSpec((C,), lambda i, r: (0,)),
            pl.BlockSpec((C,), lambda i, r: (0,)),
            pl.BlockSpec((1, C, C), lambda i, r: (r, 0, 0)),
        ],
        out_specs=[
            pl.BlockSpec((BN, CH2), lambda i, r: (r * NB + i, 0)),
            pl.BlockSpec((BN, CH2), lambda i, r: (r * NB + i, 0)),
            pl.BlockSpec((BN, C), lambda i, r: (i, 0)),
        ],
        out_shape=[
            jax.ShapeDtypeStruct((R * N, CH2), jnp.float32),
            jax.ShapeDtypeStruct((R * N, CH2), jnp.float32),
            jax.ShapeDtypeStruct((N, C), jnp.float32),
        ],
        scratch_shapes=[pltpu.VMEM((BN, C), jnp.bfloat16)],
    )(x, ln1_g, ln1_b, W_rel)


def _selfgate_body(h_ref, wself_ref, bconv_ref, wgate_ref, bgate_ref,
                   self_ref, gate_ref):
    hb = h_ref[...].astype(jnp.bfloat16)
    self_ref[...] = (
        jnp.dot(hb, wself_ref[...].astype(jnp.bfloat16),
                preferred_element_type=jnp.float32) + bconv_ref[...])
    gate_ref[...] = (
        jnp.dot(hb, wgate_ref[...].astype(jnp.bfloat16),
                preferred_element_type=jnp.float32) + bgate_ref[...])


def _selfgate(h, W_self, b_conv, W_gate, b_gate):
    return pl.pallas_call(
        _selfgate_body,
        grid=(NB,),
        in_specs=[
            pl.BlockSpec((BN, C), lambda i: (i, 0)),
            pl.BlockSpec((C, C), lambda i: (0, 0)),
            pl.BlockSpec((C,), lambda i: (0,)),
            pl.BlockSpec((C, C), lambda i: (0, 0)),
            pl.BlockSpec((C,), lambda i: (0,)),
        ],
        out_specs=[
            pl.BlockSpec((BN, C), lambda i: (i, 0)),
            pl.BlockSpec((BN, C), lambda i: (i, 0)),
        ],
        out_shape=[
            jax.ShapeDtypeStruct((N, C), jnp.float32),
            jax.ShapeDtypeStruct((N, C), jnp.float32),
        ],
    )(h, W_self, b_conv, W_gate, b_gate)


def _sc_count_body(dst_hbm, et_hbm, scale_hbm, cnt_sh, dstb, etb,
                   onesb, sidx4, scaleb, cbuf, invb, sem, q0, q1, q2, q3):
    qsem = (q0, q1, q2, q3)
    core = lax.axis_index("c")
    sub = lax.axis_index("s")
    spt = SEG_PAD // NUM_SUBCORES

    @pl.when(core == 0)
    def _():
        @pl.loop(0, spt // LANES)
        def _(i):
            cbuf[pl.ds(i * LANES, LANES)] = jnp.zeros((LANES,), jnp.float32)

        pltpu.sync_copy(cbuf, cnt_sh.at[pl.ds(sub * spt, spt)])

        @pl.loop(0, ECH // LANES)
        def _(i):
            onesb[pl.ds(i * LANES, LANES)] = jnp.full((LANES,), 1.0,
                                                      jnp.float32)

        base = sub * EPT
        pltpu.async_copy(dst_hbm.at[pl.ds(base, EPT)], dstb, sem).wait()
        pltpu.async_copy(et_hbm.at[pl.ds(base, EPT)], etb, sem).wait()
        plsc.subcore_barrier()

        def cfill(ci, b):
            @pl.loop(0, ECH // LANES)
            def _(g):
                s = pl.ds(ci * ECH + g * LANES, LANES)
                sidx4[b, pl.ds(g * LANES, LANES)] = dstb[s] * R + etb[s]

        def cwait(b):
            pltpu.make_async_copy(onesb, cnt_sh.at[sidx4.at[b]],
                                  qsem[b]).wait()

        @pl.loop(0, CHUNKS // 4)
        def _(p):
            for b in range(4):
                @pl.when(p >= 1)
                def _():
                    cwait(b)

                cfill(p * 4 + b, b)
                pltpu.async_copy(onesb, cnt_sh.at[sidx4.at[b]], qsem[b],
                                 add=True)

        for b in range(4):
            cwait(b)

        plsc.subcore_barrier()
        pltpu.sync_copy(cnt_sh.at[pl.ds(sub * spt, spt)], cbuf)

        @pl.loop(0, spt // LANES)
        def _(i):
            s = pl.ds(i * LANES, LANES)
            cbuf[s] = 1.0 / jnp.maximum(cbuf[s], 1.0)

        pltpu.sync_copy(cbuf, cnt_sh.at[pl.ds(sub * spt, spt)])
        plsc.subcore_barrier()
        pltpu.sync_copy(cnt_sh, invb)

        @pl.loop(0, K1CHUNKS)
        def _(ci):
            @pl.loop(0, K1CH // LANES)
            def _(g):
                s = pl.ds(ci * K1CH + g * LANES, LANES)
                scaleb[pl.ds(g * LANES, LANES)] = plsc.load_gather(
                    invb, [dstb[s] * R + etb[s]])

            pltpu.async_copy(scaleb,
                             scale_hbm.at[pl.ds(base + ci * K1CH, K1CH)],
                             sem).wait()


def _sc_count(dst_pad, et_pad):
    mesh = plsc.VectorSubcoreMesh(core_axis_name="c", subcore_axis_name="s")
    spt = SEG_PAD // NUM_SUBCORES
    return pl.kernel(
        _sc_count_body,
        out_type=jax.ShapeDtypeStruct((E_PAD,), jnp.float32),
        mesh=mesh,
        scratch_types=[
            pltpu.VMEM_SHARED((SEG_PAD,), jnp.float32),
            pltpu.VMEM((EPT,), jnp.int32),
            pltpu.VMEM((EPT,), jnp.int32),
            pltpu.VMEM((ECH,), jnp.float32),
            pltpu.VMEM((4, ECH), jnp.int32),
            pltpu.VMEM((K1CH,), jnp.float32),
            pltpu.VMEM((spt,), jnp.float32),
            pltpu.VMEM((SEG_PAD,), jnp.float32),
            pltpu.SemaphoreType.DMA,
            pltpu.SemaphoreType.DMA,
            pltpu.SemaphoreType.DMA,
            pltpu.SemaphoreType.DMA,
            pltpu.SemaphoreType.DMA,
        ],
        compiler_params=pltpu.CompilerParams(needs_layout_passes=False),
    )(dst_pad, et_pad)


def _sc_agg_body(hwa_hbm, hwb_hbm, src_hbm, dst_hbm, et_hbm, scale_hbm,
                 out_hbm, upd_sh, srcb, dstb, etb, scaleb, gidx4,
                 didx4, rows4, sem, g0, g1, g2, g3, s0, s1, s2, s3):
    core = lax.axis_index("c")
    sub = lax.axis_index("s")
    gsem = (g0, g1, g2, g3)
    ssem = (s0, s1, s2, s3)

    @pl.loop(0, GE)
    def _(i):
        for j in range(CH2 // LANES):
            rows4[0, i, pl.ds(j * LANES, LANES)] = jnp.zeros((LANES,),
                                                             jnp.float32)

    for k in range(-(-ZCHUNKS // NUM_SUBCORES)):
        ci = sub + k * NUM_SUBCORES

        @pl.when(ci < ZCHUNKS)
        def _():
            pltpu.sync_copy(rows4.at[0], upd_sh.at[pl.ds(ci * GE, GE)])

    plsc.subcore_barrier()

    base = sub * EPT

    def run_edges(table_hbm):
        def fill(go, slot):
            @pl.loop(0, GE // LANES)
            def _(g):
                s = pl.ds(go + g * LANES, LANES)
                d = pl.ds(g * LANES, LANES)
                gidx4[slot, d] = etb[s] * N + srcb[s]
                didx4[slot, d] = dstb[s]

        def start_g(slot):
            pltpu.async_copy(table_hbm.at[gidx4.at[slot]], rows4.at[slot],
                             gsem[slot])

        def wait_g(slot):
            pltpu.make_async_copy(table_hbm.at[gidx4.at[slot]],
                                  rows4.at[slot], gsem[slot]).wait()

        def start_s(slot):
            pltpu.async_copy(rows4.at[slot], upd_sh.at[didx4.at[slot]],
                             ssem[slot], add=True)

        def wait_s(slot):
            pltpu.make_async_copy(rows4.at[slot],
                                  upd_sh.at[didx4.at[slot]],
                                  ssem[slot]).wait()

        def mult(slot, gi):
            go = gi * GE
            rb = rows4.at[slot]

            @pl.loop(0, GE // LANES)
            def _(g):
                sv = scaleb[pl.ds(go + g * LANES, LANES)]
                for k in range(LANES):
                    sc = sv[k]
                    e = g * LANES + k
                    for j in range(CH2 // LANES):
                        s = pl.ds(j * LANES, LANES)
                        rb[e, s] = rb[e, s] * sc

        def step(gi, b):
            wait_g(b)

            @pl.when(gi + 2 < GPS)
            def _():
                sl = (b + 2) % 4

                @pl.when(gi >= 2)
                def _():
                    wait_s(sl)

                fill((gi + 2) * GE, sl)
                start_g(sl)

            mult(b, gi)
            start_s(b)

        def superchunk(off):
            c1 = pltpu.async_copy(src_hbm.at[pl.ds(off, SCH)], srcb, sem)
            c2 = pltpu.async_copy(dst_hbm.at[pl.ds(off, SCH)], dstb, sem)
            c3 = pltpu.async_copy(et_hbm.at[pl.ds(off, SCH)], etb, sem)
            c4 = pltpu.async_copy(scale_hbm.at[pl.ds(off, SCH)], scaleb,
                                  sem)
            c1.wait()
            c2.wait()
            c3.wait()
            c4.wait()

            fill(0, 0)
            start_g(0)
            fill(GE, 1)
            start_g(1)

            @pl.loop(0, GPS // 4)
            def _(p):
                for b in range(4):
                    step(p * 4 + b, b)

            for b in range(4):
                wait_s(b)

        @pl.loop(0, NSC)
        def _(ci):
            superchunk(base + ci * SCH)

    @pl.when(core == 0)
    def _():
        run_edges(hwa_hbm)

    @pl.when(core == 1)
    def _():
        run_edges(hwb_hbm)

    plsc.subcore_barrier()
    for k in range(-(-DCHUNKS // NUM_SUBCORES)):
        ci = sub + k * NUM_SUBCORES

        @pl.when(ci < DCHUNKS)
        def _():
            pltpu.sync_copy(upd_sh.at[pl.ds(ci * GE, GE)], rows4.at[0])
            pltpu.sync_copy(rows4.at[0],
                            out_hbm.at[core].at[pl.ds(ci * GE, GE)])

    tail = N - DCHUNKS * GE

    @pl.when(sub == NUM_SUBCORES - 1)
    def _():
        pltpu.sync_copy(upd_sh.at[pl.ds(DCHUNKS * GE, tail)],
                        rows4.at[0].at[pl.ds(0, tail)])
        pltpu.sync_copy(rows4.at[0].at[pl.ds(0, tail)],
                        out_hbm.at[core].at[pl.ds(DCHUNKS * GE, tail)])


def _sc_aggregate(hwa, hwb, src_pad, dst_pad, et_pad, scale):
    mesh = plsc.VectorSubcoreMesh(core_axis_name="c", subcore_axis_name="s")
    return pl.kernel(
        _sc_agg_body,
        out_type=jax.ShapeDtypeStruct((NUM_CORES, N, CH2), jnp.float32),
        mesh=mesh,
        scratch_types=[
            pltpu.VMEM_SHARED((NROWS, CH2), jnp.float32),
            pltpu.VMEM((SCH,), jnp.int32),
            pltpu.VMEM((SCH,), jnp.int32),
            pltpu.VMEM((SCH,), jnp.int32),
            pltpu.VMEM((SCH,), jnp.float32),
            pltpu.VMEM((4, GE), jnp.int32),
            pltpu.VMEM((4, GE), jnp.int32),
            pltpu.VMEM((4, GE, CH2), jnp.float32),
            pltpu.SemaphoreType.DMA,
            pltpu.SemaphoreType.DMA,
            pltpu.SemaphoreType.DMA,
            pltpu.SemaphoreType.DMA,
            pltpu.SemaphoreType.DMA,
            pltpu.SemaphoreType.DMA,
            pltpu.SemaphoreType.DMA,
            pltpu.SemaphoreType.DMA,
            pltpu.SemaphoreType.DMA,
        ],
        compiler_params=pltpu.CompilerParams(needs_layout_passes=False),
    )(hwa, hwb, src_pad, dst_pad, et_pad, scale)


def _post_body(x_ref, ua_ref, ub_ref, self_ref, gate_ref, wproj_ref,
               bproj_ref, g2_ref, b2_ref, wfc1_ref, bfc1_ref, wfc2_ref,
               bfc2_ref, out_ref):
    upd = jnp.concatenate([ua_ref[0], ub_ref[0]], axis=-1) + self_ref[...]
    gate = jax.nn.sigmoid(gate_ref[...])
    conv = gate * jax.nn.gelu(upd)
    y = x_ref[0] + jnp.dot(conv.astype(jnp.bfloat16),
                           wproj_ref[...].astype(jnp.bfloat16),
                           preferred_element_type=jnp.float32) + bproj_ref[...]
    mu = jnp.mean(y, axis=-1, keepdims=True)
    var = jnp.mean(jnp.square(y - mu), axis=-1, keepdims=True)
    h2 = (y - mu) / jnp.sqrt(var + 1e-5) * g2_ref[...] + b2_ref[...]
    f1 = jnp.dot(h2.astype(jnp.bfloat16), wfc1_ref[...].astype(jnp.bfloat16),
                 preferred_element_type=jnp.float32) + bfc1_ref[...]
    ffn = jnp.dot(
        jax.nn.gelu(f1).astype(jnp.bfloat16),
        wfc2_ref[...].astype(jnp.bfloat16),
        preferred_element_type=jnp.float32) + bfc2_ref[...]
    out_ref[0] = y + ffn


def _post(x, upd, a_self, a_gate, W_proj, b_proj, ln2_g, ln2_b,
          W_fc1, b_fc1, W_fc2, b_fc2):
    return pl.pallas_call(
        _post_body,
        grid=(NBQ,),
        in_specs=[
            pl.BlockSpec((1, BNQ, C), lambda i: (0, i, 0)),
            pl.BlockSpec((1, BNQ, CH2), lambda i: (0, i, 0)),
            pl.BlockSpec((1, BNQ, CH2), lambda i: (1, i, 0)),
            pl.BlockSpec((BNQ, C), lambda i: (i, 0)),
            pl.BlockSpec((BNQ, C), lambda i: (i, 0)),
            pl.BlockSpec((C, C), lambda i: (0, 0)),
            pl.BlockSpec((C,), lambda i: (0,)),
            pl.BlockSpec((C,), lambda i: (0,)),
            pl.BlockSpec((C,), lambda i: (0,)),
            pl.BlockSpec((C, H), lambda i: (0, 0)),
            pl.BlockSpec((H,), lambda i: (0,)),
            pl.BlockSpec((H, C), lambda i: (0, 0)),
            pl.BlockSpec((C,), lambda i: (0,)),
        ],
        out_specs=pl.BlockSpec((1, BNQ, C), lambda i: (0, i, 0)),
        out_shape=jax.ShapeDtypeStruct((1, N, C), jnp.float32),
    )(x, upd, upd, a_self, a_gate, W_proj, b_proj, ln2_g, ln2_b,
      W_fc1, b_fc1, W_fc2, b_fc2)


@jax.jit
def kernel(x, edge_index, edge_type, ln1_g, ln1_b, W_rel, W_self, b_conv,
           W_gate, b_gate, W_proj, b_proj, ln2_g, ln2_b,
           W_fc1, b_fc1, W_fc2, b_fc2):
    src = edge_index[0].astype(jnp.int32)
    dst = edge_index[1].astype(jnp.int32)
    et = edge_type.astype(jnp.int32)

    npad = E_PAD - E
    pad_i = jnp.arange(npad, dtype=jnp.int32)
    src_pad = jnp.concatenate([src, (pad_i * 127) % N])
    dst_pad = jnp.concatenate([dst, N + (pad_i % NODE_PAD)])
    et_pad = jnp.concatenate([et, jnp.zeros((npad,), jnp.int32)])

    hwa, hwb, h = _pre(x, ln1_g, ln1_b, W_rel)
    scale = _sc_count(dst_pad, et_pad)
    upd = _sc_aggregate(hwa, hwb, src_pad, dst_pad, et_pad, scale)
    a_self, a_gate = _selfgate(h, W_self, b_conv, W_gate, b_gate)
    return _post(x, upd, a_self, a_gate, W_proj, b_proj, ln2_g, ln2_b,
                 W_fc1, b_fc1, W_fc2, b_fc2)
